# Initial kernel scaffold; baseline (speedup 1.0000x reference)
import jax
import jax.numpy as jnp
from jax.experimental import pallas as pl


def kernel(x, embedding):
    raise NotImplementedError("write your pallas kernel here")


# speedup vs baseline: 4.8096x; 4.8096x over previous
import functools

import jax
import jax.numpy as jnp
from jax import lax
from jax.experimental import pallas as pl
from jax.experimental.pallas import tpu as pltpu
from jax.experimental.pallas import tpu_sc as plsc

EMBED = 32
LANES = 128
K = 8
NUM_WORKERS = 32


def _gather_body(n_rows, table, idx, out, idx_v, rows_v, sem):
    wid = lax.axis_index("s") * 2 + lax.axis_index("c")
    rows_per_w = n_rows // NUM_WORKERS
    base = wid * rows_per_w
    n_outer = rows_per_w // K

    def body(g, carry):
        row0 = base + g * K
        pltpu.sync_copy(idx.at[pl.ds(row0, K)], idx_v)
        descs = []
        for j in range(K):
            descs.append(
                pltpu.async_copy(table.at[idx_v.at[j]], rows_v.at[j], sem))
        for d in descs:
            d.wait()
        pltpu.sync_copy(rows_v, out.at[pl.ds(row0, K)])
        return carry

    lax.fori_loop(0, n_outer, body, 0)


def kernel(x, embedding):
    b0, b1 = x.shape
    total = b0 * b1
    n_rows = total // LANES
    idx = x.reshape(n_rows, LANES).astype(jnp.int32)

    gather = pl.kernel(
        functools.partial(_gather_body, n_rows),
        out_type=jax.ShapeDtypeStruct((n_rows, LANES, EMBED), jnp.float32),
        mesh=plsc.VectorSubcoreMesh(core_axis_name="c", subcore_axis_name="s"),
        scratch_types=[
            pltpu.VMEM((K, LANES), jnp.int32),
            pltpu.VMEM((K, LANES, EMBED), jnp.float32),
            pltpu.SemaphoreType.DMA,
        ],
        compiler_params=pltpu.CompilerParams(use_tc_tiling_on_sc=False),
    )
    out = gather(embedding, idx)
    return out.reshape(b0, b1, EMBED)
